# baseline (speedup 1.0000x reference)
import functools

import jax
import jax.numpy as jnp
from jax import lax
from jax.experimental import pallas as pl
from jax.experimental.pallas import tpu as pltpu
from jax.experimental.pallas import tpu_sc as plsc

BATCH = 8192
D = 2048
L = 16
NC = 2
NS = 16
NW = NC * NS
ROWS_PER_W = BATCH // NW
CHUNK = 8
N_CHUNKS = ROWS_PER_W // CHUNK
JBLKS = D // L
NBUF = 2


def _gather_chunk(in_ref, out_ref, sel_v):
  @plsc.parallel_loop(0, JBLKS, unroll=2)
  def _(j):
    col = sel_v[pl.ds(j * L, L)]
    off = j * L
    for r in range(CHUNK):
      vals = plsc.load_gather(in_ref.at[r], [col])
      out_ref.at[r][pl.ds(off, L)] = vals


def _body(tgt_hbm, perm_hbm, inv_hbm, flag_hbm, out_hbm,
          perm_v, inv_v, flag_v, sel_v, in_v, out_v, sems):
  wid = lax.axis_index("s") * NC + lax.axis_index("c")

  pltpu.sync_copy(perm_hbm, perm_v)
  pltpu.sync_copy(inv_hbm, inv_v)
  pltpu.sync_copy(flag_hbm, flag_v)
  use_inv = flag_v[...] != 0

  def sel_body(j, carry):
    p = perm_v[pl.ds(j * L, L)]
    q = inv_v[pl.ds(j * L, L)]
    sel_v[pl.ds(j * L, L)] = lax.select(use_inv, q, p)
    return carry

  lax.fori_loop(0, JBLKS, sel_body, 0)

  base = wid * ROWS_PER_W

  def chunk_slice(c):
    return pl.ds(base + c * CHUNK, CHUNK)

  h_in = [None] * NBUF
  h_out = [None] * NBUF
  h_in[0] = pltpu.async_copy(tgt_hbm.at[chunk_slice(0)], in_v.at[0],
                             sems.at[0])
  for c in range(N_CHUNKS):
    b = c % NBUF
    nb = (c + 1) % NBUF
    if c + 1 < N_CHUNKS:
      h_in[nb] = pltpu.async_copy(tgt_hbm.at[chunk_slice(c + 1)],
                                  in_v.at[nb], sems.at[nb])
    if c >= NBUF:
      h_out[b].wait()
    h_in[b].wait()
    _gather_chunk(in_v.at[b], out_v.at[b], sel_v)
    h_out[b] = pltpu.async_copy(out_v.at[b], out_hbm.at[chunk_slice(c)],
                                sems.at[NBUF + b])
  for b in range(NBUF):
    h_out[b].wait()


@functools.partial(
    pl.kernel,
    mesh=plsc.VectorSubcoreMesh(core_axis_name="c", subcore_axis_name="s"),
    out_type=jax.ShapeDtypeStruct((BATCH, D), jnp.float32),
    compiler_params=pltpu.CompilerParams(
        needs_layout_passes=False, use_tc_tiling_on_sc=False),
    scratch_types=[
        pltpu.VMEM((D,), jnp.int32),
        pltpu.VMEM((D,), jnp.int32),
        pltpu.VMEM((L,), jnp.int32),
        pltpu.VMEM((D,), jnp.int32),
        pltpu.VMEM((NBUF, CHUNK, D), jnp.float32),
        pltpu.VMEM((NBUF, CHUNK, D), jnp.float32),
        pltpu.SemaphoreType.DMA((2 * NBUF,)),
    ],
)
def _permute_sc(tgt_hbm, perm_hbm, inv_hbm, flag_hbm, out_hbm,
                perm_v, inv_v, flag_v, sel_v, in_v, out_v, sems):
  _body(tgt_hbm, perm_hbm, inv_hbm, flag_hbm, out_hbm,
        perm_v, inv_v, flag_v, sel_v, in_v, out_v, sems)


@jax.jit
def kernel(target, permutation, inv_permutation, inverse):
  flag = jnp.broadcast_to(jnp.asarray(inverse, jnp.int32), (L,))
  return _permute_sc(target, permutation, inv_permutation, flag)


# speedup vs baseline: 2.4618x; 2.4618x over previous
import functools

import jax
import jax.numpy as jnp
from jax import lax
from jax.experimental import pallas as pl
from jax.experimental.pallas import tpu as pltpu
from jax.experimental.pallas import tpu_sc as plsc

BATCH = 8192
D = 2048
L = 16
NC = 2
NS = 16
NW = NC * NS
ROWS_PER_W = BATCH // NW
CHUNK = 8
N_CHUNKS = ROWS_PER_W // CHUNK
JBLKS = D // L


def _gather_chunk(in_chunk, out_chunk, sel_v):
  @plsc.parallel_loop(0, JBLKS, unroll=2)
  def _(j):
    cols = sel_v[pl.ds(j * L, L)]
    off = j * L
    for r in range(CHUNK):
      rows = jnp.full((L,), r, jnp.int32)
      vals = plsc.load_gather(in_chunk, [rows, cols])
      out_chunk[r, pl.ds(off, L)] = vals


def _body(tgt_hbm, perm_hbm, inv_hbm, flag_hbm, out_hbm,
          perm_v, inv_v, flag_v, sel_v, in0, in1, out0, out1, sems):
  wid = lax.axis_index("s") * NC + lax.axis_index("c")

  pltpu.sync_copy(perm_hbm, perm_v)
  pltpu.sync_copy(inv_hbm, inv_v)
  pltpu.sync_copy(flag_hbm, flag_v)
  use_inv = flag_v[...] != 0

  def sel_body(j, carry):
    p = perm_v[pl.ds(j * L, L)]
    q = inv_v[pl.ds(j * L, L)]
    sel_v[pl.ds(j * L, L)] = lax.select(use_inv, q, p)
    return carry

  lax.fori_loop(0, JBLKS, sel_body, 0)

  row0 = wid * ROWS_PER_W
  in_bufs = (in0, in1)
  out_bufs = (out0, out1)

  def chunk_slice(c):
    return pl.ds(row0 + c * CHUNK, CHUNK)

  h_in = [None, None]
  h_out = [None, None]
  h_in[0] = pltpu.async_copy(tgt_hbm.at[chunk_slice(0)], in_bufs[0],
                             sems.at[0])
  for c in range(N_CHUNKS):
    b = c % 2
    nb = (c + 1) % 2
    if c + 1 < N_CHUNKS:
      h_in[nb] = pltpu.async_copy(tgt_hbm.at[chunk_slice(c + 1)],
                                  in_bufs[nb], sems.at[nb])
    if c >= 2:
      h_out[b].wait()
    h_in[b].wait()
    _gather_chunk(in_bufs[b], out_bufs[b], sel_v)
    h_out[b] = pltpu.async_copy(out_bufs[b], out_hbm.at[chunk_slice(c)],
                                sems.at[2 + b])
  for b in range(2):
    h_out[b].wait()


@functools.partial(
    pl.kernel,
    mesh=plsc.VectorSubcoreMesh(core_axis_name="c", subcore_axis_name="s"),
    out_type=jax.ShapeDtypeStruct((BATCH, D), jnp.float32),
    compiler_params=pltpu.CompilerParams(
        needs_layout_passes=False, use_tc_tiling_on_sc=True),
    scratch_types=[
        pltpu.VMEM((D,), jnp.int32),
        pltpu.VMEM((D,), jnp.int32),
        pltpu.VMEM((L,), jnp.int32),
        pltpu.VMEM((D,), jnp.int32),
        pltpu.VMEM((CHUNK, D), jnp.float32),
        pltpu.VMEM((CHUNK, D), jnp.float32),
        pltpu.VMEM((CHUNK, D), jnp.float32),
        pltpu.VMEM((CHUNK, D), jnp.float32),
        pltpu.SemaphoreType.DMA((4,)),
    ],
)
def _permute_sc(tgt_hbm, perm_hbm, inv_hbm, flag_hbm, out_hbm,
                perm_v, inv_v, flag_v, sel_v, in0, in1, out0, out1, sems):
  _body(tgt_hbm, perm_hbm, inv_hbm, flag_hbm, out_hbm,
        perm_v, inv_v, flag_v, sel_v, in0, in1, out0, out1, sems)


@jax.jit
def kernel(target, permutation, inv_permutation, inverse):
  flag = jnp.broadcast_to(jnp.asarray(inverse, jnp.int32), (L,))
  return _permute_sc(target, permutation, inv_permutation, flag)


# speedup vs baseline: 2.5449x; 1.0338x over previous
import functools

import jax
import jax.numpy as jnp
from jax import lax
from jax.experimental import pallas as pl
from jax.experimental.pallas import tpu as pltpu
from jax.experimental.pallas import tpu_sc as plsc

BATCH = 8192
D = 2048
L = 16
NC = 2
NS = 16
NW = NC * NS
ROWS_PER_W = BATCH // NW
CHUNK = 8
N_CHUNKS = ROWS_PER_W // CHUNK
JBLKS = D // L
NBUF = 3


def _gather_chunk(in_chunk, out_chunk, sel_v, row_splats):
  @plsc.parallel_loop(0, JBLKS, unroll=4)
  def _(j):
    cols = sel_v[pl.ds(j * L, L)]
    off = j * L
    for r in range(CHUNK):
      vals = plsc.load_gather(in_chunk, [row_splats[r], cols])
      out_chunk[r, pl.ds(off, L)] = vals


def _body(tgt_hbm, perm_hbm, inv_hbm, flag_hbm, out_hbm,
          perm_v, inv_v, flag_v, sel_v, in_bufs, out_bufs, sems):
  wid = lax.axis_index("s") * NC + lax.axis_index("c")

  pltpu.sync_copy(perm_hbm, perm_v)
  pltpu.sync_copy(inv_hbm, inv_v)
  pltpu.sync_copy(flag_hbm, flag_v)
  use_inv = flag_v[...] != 0

  def sel_body(j, carry):
    p = perm_v[pl.ds(j * L, L)]
    q = inv_v[pl.ds(j * L, L)]
    sel_v[pl.ds(j * L, L)] = lax.select(use_inv, q, p)
    return carry

  lax.fori_loop(0, JBLKS, sel_body, 0)

  row_splats = [jnp.full((L,), r, jnp.int32) for r in range(CHUNK)]
  row0 = wid * ROWS_PER_W

  def chunk_slice(c):
    return pl.ds(row0 + c * CHUNK, CHUNK)

  h_in = [None] * NBUF
  h_out = [None] * NBUF
  for p in range(NBUF - 1):
    h_in[p] = pltpu.async_copy(tgt_hbm.at[chunk_slice(p)], in_bufs[p],
                               sems.at[p])
  for c in range(N_CHUNKS):
    b = c % NBUF
    nb = (c + NBUF - 1) % NBUF
    if c + NBUF - 1 < N_CHUNKS:
      h_in[nb] = pltpu.async_copy(tgt_hbm.at[chunk_slice(c + NBUF - 1)],
                                  in_bufs[nb], sems.at[nb])
    if c >= NBUF:
      h_out[b].wait()
    h_in[b].wait()
    _gather_chunk(in_bufs[b], out_bufs[b], sel_v, row_splats)
    h_out[b] = pltpu.async_copy(out_bufs[b], out_hbm.at[chunk_slice(c)],
                                sems.at[NBUF + b])
  for b in range(NBUF):
    h_out[b].wait()


@functools.partial(
    pl.kernel,
    mesh=plsc.VectorSubcoreMesh(core_axis_name="c", subcore_axis_name="s"),
    out_type=jax.ShapeDtypeStruct((BATCH, D), jnp.float32),
    compiler_params=pltpu.CompilerParams(
        needs_layout_passes=False, use_tc_tiling_on_sc=True),
    scratch_types=[
        pltpu.VMEM((D,), jnp.int32),
        pltpu.VMEM((D,), jnp.int32),
        pltpu.VMEM((L,), jnp.int32),
        pltpu.VMEM((D,), jnp.int32),
    ] + [pltpu.VMEM((CHUNK, D), jnp.float32) for _ in range(2 * NBUF)] + [
        pltpu.SemaphoreType.DMA((2 * NBUF,)),
    ],
)
def _permute_sc(tgt_hbm, perm_hbm, inv_hbm, flag_hbm, out_hbm,
                perm_v, inv_v, flag_v, sel_v, *bufs_and_sems):
  bufs = bufs_and_sems[:2 * NBUF]
  sems = bufs_and_sems[2 * NBUF]
  _body(tgt_hbm, perm_hbm, inv_hbm, flag_hbm, out_hbm,
        perm_v, inv_v, flag_v, sel_v, bufs[:NBUF], bufs[NBUF:], sems)


@jax.jit
def kernel(target, permutation, inv_permutation, inverse):
  flag = jnp.broadcast_to(jnp.asarray(inverse, jnp.int32), (L,))
  return _permute_sc(target, permutation, inv_permutation, flag)


# speedup vs baseline: 2.6136x; 1.0270x over previous
import functools

import jax
import jax.numpy as jnp
from jax import lax
from jax.experimental import pallas as pl
from jax.experimental.pallas import tpu as pltpu
from jax.experimental.pallas import tpu_sc as plsc

BATCH = 8192
D = 2048
L = 16
NC = 2
NS = 16
NW = NC * NS
ROWS_PER_W = BATCH // NW
CHUNK = 8
N_CHUNKS = ROWS_PER_W // CHUNK
JBLKS = D // L
NBUF = 3


def _gather_chunk(in_chunk, out_chunk, sel_v, row_splats):
  @plsc.parallel_loop(0, JBLKS, unroll=4)
  def _(j):
    cols = sel_v[pl.ds(j * L, L)]
    off = j * L
    for r in range(CHUNK):
      vals = plsc.load_gather(in_chunk, [row_splats[r], cols])
      out_chunk[r, pl.ds(off, L)] = vals


def _body(tgt_hbm, perm_hbm, inv_hbm, flag_hbm, out_hbm,
          perm_v, inv_v, flag_v, sel_v, in_bufs, out_bufs, sems):
  wid = lax.axis_index("s") * NC + lax.axis_index("c")
  row0 = wid * ROWS_PER_W

  def chunk_slice(c):
    return pl.ds(row0 + c * CHUNK, CHUNK)

  h_in = [None] * NBUF
  for p in range(NBUF - 1):
    h_in[p] = pltpu.async_copy(tgt_hbm.at[chunk_slice(p)], in_bufs[p],
                               sems.at[p])

  pltpu.sync_copy(perm_hbm, perm_v)
  pltpu.sync_copy(inv_hbm, inv_v)
  pltpu.sync_copy(flag_hbm, flag_v)
  use_inv = flag_v[...] != 0

  def sel_body(j, carry):
    p = perm_v[pl.ds(j * L, L)]
    q = inv_v[pl.ds(j * L, L)]
    sel_v[pl.ds(j * L, L)] = lax.select(use_inv, q, p)
    return carry

  lax.fori_loop(0, JBLKS, sel_body, 0)

  row_splats = [jnp.full((L,), r, jnp.int32) for r in range(CHUNK)]
  h_out = [None] * NBUF
  for c in range(N_CHUNKS):
    b = c % NBUF
    nb = (c + NBUF - 1) % NBUF
    if c + NBUF - 1 < N_CHUNKS:
      h_in[nb] = pltpu.async_copy(tgt_hbm.at[chunk_slice(c + NBUF - 1)],
                                  in_bufs[nb], sems.at[nb])
    if c >= NBUF:
      h_out[b].wait()
    h_in[b].wait()
    _gather_chunk(in_bufs[b], out_bufs[b], sel_v, row_splats)
    h_out[b] = pltpu.async_copy(out_bufs[b], out_hbm.at[chunk_slice(c)],
                                sems.at[NBUF + b])
  for b in range(NBUF):
    h_out[b].wait()


@functools.partial(
    pl.kernel,
    mesh=plsc.VectorSubcoreMesh(core_axis_name="c", subcore_axis_name="s"),
    out_type=jax.ShapeDtypeStruct((BATCH, D), jnp.float32),
    compiler_params=pltpu.CompilerParams(
        needs_layout_passes=False, use_tc_tiling_on_sc=True),
    scratch_types=[
        pltpu.VMEM((D,), jnp.int32),
        pltpu.VMEM((D,), jnp.int32),
        pltpu.VMEM((L,), jnp.int32),
        pltpu.VMEM((D,), jnp.int32),
    ] + [pltpu.VMEM((CHUNK, D), jnp.float32) for _ in range(2 * NBUF)] + [
        pltpu.SemaphoreType.DMA((2 * NBUF,)),
    ],
)
def _permute_sc(tgt_hbm, perm_hbm, inv_hbm, flag_hbm, out_hbm,
                perm_v, inv_v, flag_v, sel_v, *bufs_and_sems):
  bufs = bufs_and_sems[:2 * NBUF]
  sems = bufs_and_sems[2 * NBUF]
  _body(tgt_hbm, perm_hbm, inv_hbm, flag_hbm, out_hbm,
        perm_v, inv_v, flag_v, sel_v, bufs[:NBUF], bufs[NBUF:], sems)


@jax.jit
def kernel(target, permutation, inv_permutation, inverse):
  flag = jnp.broadcast_to(jnp.asarray(inverse, jnp.int32), (L,))
  return _permute_sc(target, permutation, inv_permutation, flag)
